# baseline (speedup 1.0000x reference)
import functools

import jax
import jax.numpy as jnp
from jax import lax
from jax.experimental import pallas as pl
from jax.experimental.pallas import tpu as pltpu
from jax.experimental.pallas import tpu_sc as plsc

_EPS = 1e-7

_NC = 2
_NS = 16
_NW = _NC * _NS
_CHUNK_ROWS = 16
_DPC = _CHUNK_ROWS * 364
_TPC = _CHUNK_ROWS * 91


def _tc_scores_kernel(ts_ref, os_ref, out_ref, acc_ref):
    i = pl.program_id(0) * pl.num_programs(1) + pl.program_id(1)
    g = pl.num_programs(0) * pl.num_programs(1)

    ts = ts_ref[0]
    osc = os_ref[0]
    s = jnp.sum(osc, axis=1, keepdims=True)
    p = jnp.clip(osc / s, _EPS, 1.0 - _EPS)
    ce_c = jnp.sum(ts * jnp.log(p))
    col = lax.broadcasted_iota(jnp.int32, ts.shape, 1)
    pos_c = jnp.sum(ts * (col >= 1).astype(jnp.float32))

    @pl.when(i == 0)
    def _init():
        acc_ref[0] = 0.0
        acc_ref[1] = 0.0

    acc_ref[0] += ce_c
    acc_ref[1] += pos_c

    @pl.when(i == g - 1)
    def _fin():
        out_ref[...] = jnp.concatenate(
            [jnp.reshape(acc_ref[0], (1, 1)), jnp.reshape(acc_ref[1], (1, 1))],
            axis=1)


def _sc_reg_kernel(td_hbm, od_hbm, ts_hbm, out_hbm,
                   td_v, od_v, ts_v, acc_v, n_chunks):
    cid = lax.axis_index("c")
    sid = lax.axis_index("s")
    wid = sid * _NC + cid

    iota = lax.iota(jnp.int32, 16)
    exp_idx = iota // 4

    rem = n_chunks % _NW
    my_n = n_chunks // _NW + (wid < rem).astype(jnp.int32)

    def chunk_body(t, acc):
        ch = wid + _NW * t
        pltpu.sync_copy(td_hbm.at[pl.ds(ch * _DPC, _DPC)], td_v)
        pltpu.sync_copy(od_hbm.at[pl.ds(ch * _DPC, _DPC)], od_v)
        pltpu.sync_copy(ts_hbm.at[pl.ds(ch * _TPC, _TPC)], ts_v)

        for v in range(_DPC // 16):
            tdq = td_v[pl.ds(16 * v, 16)]
            odq = od_v[pl.ds(16 * v, 16)]
            x = tdq - odq
            ax = jnp.abs(x)
            mn = jnp.minimum(ax, 1.0)
            scq = plsc.load_gather(ts_v, [jnp.int32(4 * v) + exp_idx])
            scq = jnp.where(scq == 1.0, 0.5, 0.0)
            k0 = (-4 * v) % 91
            if k0 < 4:
                scq = jnp.where(exp_idx == k0, 0.0, scq)
            acc = acc + scq * (mn * (ax + ax - mn))
        return acc

    acc = lax.fori_loop(0, my_n, chunk_body, jnp.zeros((16,), jnp.float32))
    acc_v[...] = acc
    pltpu.sync_copy(acc_v, out_hbm.at[wid])


def _sc_reg_call(td_flat, od_flat, ts_flat, n_chunks):
    mesh = plsc.VectorSubcoreMesh(core_axis_name="c", subcore_axis_name="s")
    kfn = functools.partial(
        pl.kernel,
        mesh=mesh,
        compiler_params=pltpu.CompilerParams(needs_layout_passes=False),
        out_type=jax.ShapeDtypeStruct((_NW, 16), jnp.float32),
        scratch_types=[
            pltpu.VMEM((_DPC,), jnp.float32),
            pltpu.VMEM((_DPC,), jnp.float32),
            pltpu.VMEM((_TPC,), jnp.float32),
            pltpu.VMEM((16,), jnp.float32),
        ],
    )(functools.partial(_sc_reg_kernel, n_chunks=n_chunks))
    return kfn(td_flat, od_flat, ts_flat)


@jax.jit
def kernel(target_deltas, target_scores, output_deltas, output_scores):
    b, n, c = target_scores.shape
    rows = b * n

    blk = 3000
    grid = (b, n // blk)
    ce_pos = pl.pallas_call(
        _tc_scores_kernel,
        grid=grid,
        in_specs=[
            pl.BlockSpec((1, blk, c), lambda i, j: (i, j, 0)),
            pl.BlockSpec((1, blk, c), lambda i, j: (i, j, 0)),
        ],
        out_specs=pl.BlockSpec((1, 2), lambda i, j: (0, 0)),
        out_shape=jax.ShapeDtypeStruct((1, 2), jnp.float32),
        scratch_shapes=[pltpu.SMEM((2,), jnp.float32)],
    )(target_scores, output_scores)

    n_chunks = rows // _CHUNK_ROWS
    regp = _sc_reg_call(
        target_deltas.reshape(-1),
        output_deltas.reshape(-1),
        target_scores.reshape(-1),
        n_chunks,
    )

    cls_loss = -ce_pos[0, 0] / rows
    reg_loss = jnp.sum(regp) / jnp.maximum(_EPS, ce_pos[0, 1])
    return cls_loss + reg_loss


# speedup vs baseline: 2.0282x; 2.0282x over previous
import functools

import jax
import jax.numpy as jnp
from jax import lax
from jax.experimental import pallas as pl
from jax.experimental.pallas import tpu as pltpu
from jax.experimental.pallas import tpu_sc as plsc

_EPS = 1e-7

_NC = 2
_NS = 16
_NW = _NC * _NS
_CHUNK_ROWS = 16
_DPC = _CHUNK_ROWS * 364
_TPC = _CHUNK_ROWS * 91
_CPB = 6000 // _CHUNK_ROWS


def _tc_scores_kernel(ts_ref, os_ref, out_ref, acc_ref):
    i = pl.program_id(0) * pl.num_programs(1) + pl.program_id(1)
    g = pl.num_programs(0) * pl.num_programs(1)

    ts = ts_ref[0]
    osc = os_ref[0]
    s = jnp.sum(osc, axis=1, keepdims=True)
    p = jnp.clip(osc / s, _EPS, 1.0 - _EPS)
    ce_c = jnp.sum(ts * jnp.log(p))
    col = lax.broadcasted_iota(jnp.int32, ts.shape, 1)
    pos_c = jnp.sum(ts * (col >= 1).astype(jnp.float32))

    @pl.when(i == 0)
    def _init():
        acc_ref[0] = 0.0
        acc_ref[1] = 0.0

    acc_ref[0] += ce_c
    acc_ref[1] += pos_c

    @pl.when(i == g - 1)
    def _fin():
        out_ref[...] = jnp.concatenate(
            [jnp.reshape(acc_ref[0], (1, 1)), jnp.reshape(acc_ref[1], (1, 1))],
            axis=1)


def _sc_reg_kernel(td_hbm, od_hbm, ts_hbm, out_hbm,
                   td_v, od_v, ts_v, acc_v, n_chunks):
    cid = lax.axis_index("c")
    sid = lax.axis_index("s")
    wid = sid * _NC + cid

    iota = lax.iota(jnp.int32, 16)
    exp_idx = iota // 4

    rem = n_chunks % _NW
    my_n = n_chunks // _NW + (wid < rem).astype(jnp.int32)

    def chunk_body(t, acc):
        ch = wid + _NW * t
        b = ch // _CPB
        r0 = (ch % _CPB) * _CHUNK_ROWS
        pltpu.sync_copy(td_hbm.at[b, pl.ds(r0, _CHUNK_ROWS), :], td_v)
        pltpu.sync_copy(od_hbm.at[b, pl.ds(r0, _CHUNK_ROWS), :], od_v)
        pltpu.sync_copy(ts_hbm.at[b, pl.ds(r0, _CHUNK_ROWS), :], ts_v)

        row = jnp.zeros((16,), jnp.int32)
        col = iota
        for v in range(_DPC // 16):
            tdq = plsc.load_gather(td_v, [row, col])
            odq = plsc.load_gather(od_v, [row, col])
            x = tdq - odq
            ax = jnp.abs(x)
            mn = jnp.minimum(ax, 1.0)
            scq = plsc.load_gather(ts_v, [row, col >> 2])
            scq = jnp.where(scq == 1.0, 0.5, 0.0)
            bg = [l for l in range(16) if (16 * v + l) % 364 < 4]
            if bg:
                scq = jnp.where((iota >= bg[0]) & (iota <= bg[-1]), 0.0, scq)
            acc = acc + scq * (mn * (ax + ax - mn))
            colp = col + 16
            w = colp >= 364
            col = jnp.where(w, colp - 364, colp)
            row = jnp.where(w, row + 1, row)
        return acc

    acc = lax.fori_loop(0, my_n, chunk_body, jnp.zeros((16,), jnp.float32))
    acc_v[...] = acc
    pltpu.sync_copy(acc_v, out_hbm.at[wid])


def _sc_reg_call(td_flat, od_flat, ts_flat, n_chunks):
    mesh = plsc.VectorSubcoreMesh(core_axis_name="c", subcore_axis_name="s")
    kfn = functools.partial(
        pl.kernel,
        mesh=mesh,
        compiler_params=pltpu.CompilerParams(needs_layout_passes=False),
        out_type=jax.ShapeDtypeStruct((_NW, 16), jnp.float32),
        scratch_types=[
            pltpu.VMEM((_CHUNK_ROWS, 364), jnp.float32),
            pltpu.VMEM((_CHUNK_ROWS, 364), jnp.float32),
            pltpu.VMEM((_CHUNK_ROWS, 91), jnp.float32),
            pltpu.VMEM((16,), jnp.float32),
        ],
    )(functools.partial(_sc_reg_kernel, n_chunks=n_chunks))
    return kfn(td_flat, od_flat, ts_flat)


@jax.jit
def kernel(target_deltas, target_scores, output_deltas, output_scores):
    b, n, c = target_scores.shape
    rows = b * n

    blk = 3000
    grid = (b, n // blk)
    ce_pos = pl.pallas_call(
        _tc_scores_kernel,
        grid=grid,
        in_specs=[
            pl.BlockSpec((1, blk, c), lambda i, j: (i, j, 0)),
            pl.BlockSpec((1, blk, c), lambda i, j: (i, j, 0)),
        ],
        out_specs=pl.BlockSpec((1, 2), lambda i, j: (0, 0)),
        out_shape=jax.ShapeDtypeStruct((1, 2), jnp.float32),
        scratch_shapes=[pltpu.SMEM((2,), jnp.float32)],
    )(target_scores, output_scores)

    n_chunks = rows // _CHUNK_ROWS
    regp = _sc_reg_call(
        target_deltas,
        output_deltas,
        target_scores,
        n_chunks,
    )

    cls_loss = -ce_pos[0, 0] / rows
    reg_loss = jnp.sum(regp) / jnp.maximum(_EPS, ce_pos[0, 1])
    return cls_loss + reg_loss


# speedup vs baseline: 4.7593x; 2.3465x over previous
import jax
import jax.numpy as jnp
from jax import lax
from jax.experimental import pallas as pl
from jax.experimental.pallas import tpu as pltpu

_EPS = 1e-7


def _loss_kernel(td_ref, ts_ref, od_ref, os_ref, out_ref, acc_ref):
    i = pl.program_id(0) * pl.num_programs(1) + pl.program_id(1)
    g = pl.num_programs(0) * pl.num_programs(1)

    ts = ts_ref[0]
    osc = os_ref[0]
    s = jnp.sum(osc, axis=1, keepdims=True)
    p = jnp.clip(osc / s, _EPS, 1.0 - _EPS)
    ce_c = jnp.sum(ts * jnp.log(p))

    col = lax.broadcasted_iota(jnp.int32, ts.shape, 1)
    tsf = ts * (col >= 1).astype(jnp.float32)
    pos_c = jnp.sum(tsf)

    x = td_ref[0] - od_ref[0]
    ax = jnp.abs(x)
    mn = jnp.minimum(ax, 1.0)
    h = mn * (ax - 0.5 * mn)

    c4 = ts.shape[1]
    rj = lax.broadcasted_iota(jnp.int32, (4 * c4, c4), 0) // 4
    cj = lax.broadcasted_iota(jnp.int32, (4 * c4, c4), 1)
    m = (rj == cj).astype(jnp.float32)
    h4 = lax.dot(h, m, preferred_element_type=jnp.float32)
    reg_c = jnp.sum(tsf * h4)

    @pl.when(i == 0)
    def _init():
        acc_ref[0] = 0.0
        acc_ref[1] = 0.0
        acc_ref[2] = 0.0

    acc_ref[0] += ce_c
    acc_ref[1] += pos_c
    acc_ref[2] += reg_c

    @pl.when(i == g - 1)
    def _fin():
        out_ref[...] = jnp.concatenate(
            [jnp.reshape(acc_ref[0], (1, 1)),
             jnp.reshape(acc_ref[1], (1, 1)),
             jnp.reshape(acc_ref[2], (1, 1))], axis=1)


@jax.jit
def kernel(target_deltas, target_scores, output_deltas, output_scores):
    b, n, c = target_scores.shape
    rows = b * n

    blk = 3000
    grid = (b, n // blk)
    acc = pl.pallas_call(
        _loss_kernel,
        grid=grid,
        in_specs=[
            pl.BlockSpec((1, blk, 4 * c), lambda i, j: (i, j, 0)),
            pl.BlockSpec((1, blk, c), lambda i, j: (i, j, 0)),
            pl.BlockSpec((1, blk, 4 * c), lambda i, j: (i, j, 0)),
            pl.BlockSpec((1, blk, c), lambda i, j: (i, j, 0)),
        ],
        out_specs=pl.BlockSpec((1, 3), lambda i, j: (0, 0)),
        out_shape=jax.ShapeDtypeStruct((1, 3), jnp.float32),
        scratch_shapes=[pltpu.SMEM((3,), jnp.float32)],
    )(target_deltas, target_scores, output_deltas, output_scores)

    cls_loss = -acc[0, 0] / rows
    reg_loss = acc[0, 2] / jnp.maximum(_EPS, acc[0, 1])
    return cls_loss + reg_loss
